# Initial kernel scaffold; baseline (speedup 1.0000x reference)
import jax
import jax.numpy as jnp
from jax.experimental import pallas as pl


def kernel(pc, params):
    raise NotImplementedError("write your pallas kernel here")


# speedup vs baseline: 7.8608x; 7.8608x over previous
import functools
import math

import jax
import jax.numpy as jnp
from jax import lax
from jax.experimental import pallas as pl
from jax.experimental.pallas import tpu as pltpu
from jax.experimental.pallas import tpu_sc as plsc

_B = 16
_N1 = 4096
_NP1 = 512
_NP2 = 128
_MLPS1 = [[3, 32, 32, 64], [3, 64, 64, 128], [3, 64, 96, 128]]
_MLPS2 = [[323, 64, 64, 128], [323, 128, 128, 256], [323, 128, 128, 256]]
_SA = [643, 256, 512, 1024]
_RADII1 = [0.1, 0.2, 0.4]
_NS1 = [16, 32, 128]
_RADII2 = [0.2, 0.4, 0.8]
_NS2 = [32, 64, 128]
_EPS_SCALE = 1.0 / math.sqrt(1.0 + 1e-5)


def _fps_body(npoint, n, x_ref, cx_ref, cy_ref, cz_ref):
    X = x_ref[0:_B, :]
    Y = x_ref[_B:2 * _B, :]
    Z = x_ref[2 * _B:3 * _B, :]
    iota = lax.broadcasted_iota(jnp.int32, (_B, n), 1)

    def step(t, carry):
        dists, fx, fy, fz = carry
        cx_ref[pl.ds(t, 1), :] = fx.reshape(1, _B)
        cy_ref[pl.ds(t, 1), :] = fy.reshape(1, _B)
        cz_ref[pl.ds(t, 1), :] = fz.reshape(1, _B)
        d = (X - fx[:, None]) ** 2 + (Y - fy[:, None]) ** 2 + (Z - fz[:, None]) ** 2
        dists = jnp.minimum(dists, d)
        m = jnp.max(dists, axis=1, keepdims=True)
        nxt = jnp.min(jnp.where(dists == m, iota, n), axis=1)
        sel = (iota == nxt[:, None]).astype(jnp.float32)
        fx = jnp.sum(X * sel, axis=1)
        fy = jnp.sum(Y * sel, axis=1)
        fz = jnp.sum(Z * sel, axis=1)
        return dists, fx, fy, fz

    init = (jnp.full((_B, n), 1e10, jnp.float32), X[:, 0], Y[:, 0], Z[:, 0])
    lax.fori_loop(0, npoint, step, init)


def _fps(xyz_t, npoint):
    n = xyz_t.shape[1]
    out = jax.ShapeDtypeStruct((npoint, _B), jnp.float32)
    cx, cy, cz = pl.pallas_call(
        functools.partial(_fps_body, npoint, n),
        out_shape=(out, out, out),
    )(xyz_t)
    new_xyz = jnp.stack([cx.T, cy.T, cz.T], axis=-1)
    cent3 = jnp.stack([cx.T.reshape(-1), cy.T.reshape(-1), cz.T.reshape(-1)])
    xyz3 = jnp.stack([cx.T, cy.T, cz.T])
    return new_xyz, cent3, xyz3


_NC = 2
_NSUB = 16
_NW = _NC * _NSUB
_NSP1 = [ns + 16 for ns in _NS1]
_NSP2 = [ns + 16 for ns in _NS2]


def _bf16r(x):
    bits = lax.bitcast_convert_type(x, jnp.int32)
    r = (bits + jnp.int32(0x7FFF) + ((bits >> 16) & 1)) & jnp.int32(-65536)
    return lax.bitcast_convert_type(r, jnp.float32)


def _bq_scan(xyzv, bxyzv, pnv, cx, cy, cz, r2, ns, nsp, nchunks,
             relbuf, idxbuf, gbase):
    lanes = lax.iota(jnp.int32, 16)
    cn2 = cx * cx + cy * cy + cz * cz
    bcx, bcy, bcz = _bf16r(cx), _bf16r(cy), _bf16r(cz)

    def cond(st):
        cnt, ch = st
        return jnp.logical_and(cnt < ns, ch < nchunks)

    def body(st):
        cnt, ch = st
        base = ch * 16
        px = xyzv[0][pl.ds(base, 16)]
        py = xyzv[1][pl.ds(base, 16)]
        pz = xyzv[2][pl.ds(base, 16)]
        bpx = bxyzv[0][pl.ds(base, 16)]
        bpy = bxyzv[1][pl.ds(base, 16)]
        bpz = bxyzv[2][pl.ds(base, 16)]
        pn = pnv[pl.ds(base, 16)]
        d2 = (cn2 + pn) - 2.0 * (bcx * bpx + bcy * bpy + bcz * bpz)
        mask = d2 <= r2
        cs = plsc.cumsum(mask.astype(jnp.int32))
        slot = cnt + cs - 1
        pos = slot * 3
        plsc.store_scatter(relbuf, [pos], px - cx, mask=mask)
        plsc.store_scatter(relbuf, [pos + 1], py - cy, mask=mask)
        plsc.store_scatter(relbuf, [pos + 2], pz - cz, mask=mask)
        if idxbuf is not None:
            plsc.store_scatter(idxbuf, [slot], gbase + base + lanes,
                               mask=mask)
        return cnt + cs[15], ch + jnp.int32(1)

    cnt, _ = lax.while_loop(cond, body, (jnp.int32(0), jnp.int32(0)))
    return cnt


def _centroid_scalar(cv, i):
    i16 = (i // 16) * 16
    lane = i - i16
    v = cv[pl.ds(i16, 16)]
    sel = lax.iota(jnp.int32, 16) == lane
    return jnp.max(jnp.where(sel, v, jnp.float32(-jnp.inf)))


def _store_scalar_i32(ref, i, val):
    lanes = lax.iota(jnp.int32, 16)
    plsc.store_scatter(ref, [jnp.full((16,), jnp.int32(0), jnp.int32) + i],
                       jnp.full((16,), jnp.int32(0), jnp.int32) + val,
                       mask=lanes == 0)


def _precompute_rounded(xv, yv, zv, bxv, byv, bzv, pnv, n):
    for q in range(n // 16):
        sl = pl.ds(q * 16, 16)
        px, py, pz = xv[sl], yv[sl], zv[sl]
        bxv[sl] = _bf16r(px)
        byv[sl] = _bf16r(py)
        bzv[sl] = _bf16r(pz)
        pnv[sl] = px * px + py * py + pz * pz


def _zero_f32(ref, n):
    z = jnp.zeros((16,), jnp.float32)
    for q in range(n // 16):
        ref[pl.ds(q * 16, 16)] = z


def _bq1_body(x_hbm, cent_hbm, rel1_o, rel2_o, rel3_o, cnt_o,
              xv, yv, zv, bxv, byv, bzv, pnv, cxv, cyv, czv,
              rb1, rb2, rb3, c1v, c2v, c3v, rs1, rs2, rs3):
    branches = list(zip(_RADII1, _NS1, _NSP1))
    bs = _B * _NP1
    bpw = bs // _NW
    nchunks = _N1 // 16
    wid = lax.axis_index("s") * _NC + lax.axis_index("c")
    b = wid // (_NP1 // bpw)
    for c, dst in enumerate([xv, yv, zv]):
        pltpu.sync_copy(x_hbm.at[pl.ds((c * _B + b) * _N1, _N1)], dst)
    for c, dst in enumerate([cxv, cyv, czv]):
        pltpu.sync_copy(cent_hbm.at[pl.ds(c * bs + wid * bpw, bpw)], dst)
    _precompute_rounded(xv, yv, zv, bxv, byv, bzv, pnv, _N1)
    relbufs = [rb1, rb2, rb3]
    relsems = [rs1, rs2, rs3]
    for j, (r, ns, nsp) in enumerate(branches):
        _zero_f32(relbufs[j], 3 * nsp)
    rel_outs = [rel1_o, rel2_o, rel3_o]
    cnt_bufs = [c1v, c2v, c3v]

    lx = xv[pl.ds(_N1 - 16, 16)][15]
    ly = yv[pl.ds(_N1 - 16, 16)][15]
    lz = zv[pl.ds(_N1 - 16, 16)][15]

    def per_ball(i, carry):
        cx = _centroid_scalar(cxv, i)
        cy = _centroid_scalar(cyv, i)
        cz = _centroid_scalar(czv, i)
        for j, (r, ns, nsp) in enumerate(branches):
            relbuf = relbufs[j]

            @pl.when(i > 0)
            def _():
                pltpu.make_async_copy(
                    relbuf.at[pl.ds(0, 3 * nsp)],
                    rel_outs[j].at[pl.ds((wid * bpw + i - 1) * 3 * nsp,
                                         3 * nsp)],
                    relsems[j]).wait()

            cnt = _bq_scan([xv, yv, zv], [bxv, byv, bzv], pnv, cx, cy, cz,
                           jnp.float32(r * r), ns, nsp, nchunks,
                           relbuf, None, 0)

            @pl.when(cnt == 0)
            def _():
                lane = lax.iota(jnp.int32, 16)
                v = jnp.where(lane == 0, lx - cx,
                              jnp.where(lane == 1, ly - cy, lz - cz))
                relbuf[pl.ds(0, 16)] = v

            _store_scalar_i32(cnt_bufs[j], i,
                              jnp.maximum(jnp.minimum(cnt, ns), 1))
            pltpu.async_copy(relbuf.at[pl.ds(0, 3 * nsp)],
                             rel_outs[j].at[pl.ds((wid * bpw + i) * 3 * nsp,
                                                  3 * nsp)],
                             relsems[j])
        return carry

    lax.fori_loop(0, bpw, per_ball, 0)
    for j, (r, ns, nsp) in enumerate(branches):
        pltpu.make_async_copy(
            relbufs[j].at[pl.ds(0, 3 * nsp)],
            rel_outs[j].at[pl.ds((wid * bpw + bpw - 1) * 3 * nsp, 3 * nsp)],
            relsems[j]).wait()
    for j in range(3):
        pltpu.sync_copy(cnt_bufs[j], cnt_o.at[pl.ds(j * bs + wid * bpw, bpw)])


def _ballquery1(xyz_flat, cent_flat):
    bs = _B * _NP1
    bpw = bs // _NW
    mesh = plsc.VectorSubcoreMesh(core_axis_name="c", subcore_axis_name="s")
    outs = tuple(jax.ShapeDtypeStruct((bs * 3 * nsp,), jnp.float32)
                 for nsp in _NSP1) + (jax.ShapeDtypeStruct((3 * bs,), jnp.int32),)
    k = pl.kernel(
        _bq1_body,
        out_type=outs,
        mesh=mesh,
        compiler_params=pltpu.CompilerParams(needs_layout_passes=False),
        scratch_types=[
            pltpu.VMEM((_N1,), jnp.float32),
            pltpu.VMEM((_N1,), jnp.float32),
            pltpu.VMEM((_N1,), jnp.float32),
            pltpu.VMEM((_N1,), jnp.float32),
            pltpu.VMEM((_N1,), jnp.float32),
            pltpu.VMEM((_N1,), jnp.float32),
            pltpu.VMEM((_N1,), jnp.float32),
            pltpu.VMEM((bpw,), jnp.float32),
            pltpu.VMEM((bpw,), jnp.float32),
            pltpu.VMEM((bpw,), jnp.float32),
            pltpu.VMEM((3 * _NSP1[0],), jnp.float32),
            pltpu.VMEM((3 * _NSP1[1],), jnp.float32),
            pltpu.VMEM((3 * _NSP1[2],), jnp.float32),
            pltpu.VMEM((bpw,), jnp.int32),
            pltpu.VMEM((bpw,), jnp.int32),
            pltpu.VMEM((bpw,), jnp.int32),
            pltpu.SemaphoreType.DMA,
            pltpu.SemaphoreType.DMA,
            pltpu.SemaphoreType.DMA,
        ],
    )
    return k(xyz_flat, cent_flat)


_D2 = [m[1] for m in _MLPS2]
_DG = 128


def _bq2_body(x_hbm, cent_hbm, f1_hbm, f2_hbm, f3_hbm,
              rel1_o, rel2_o, rel3_o, g1_o, g2_o, g3_o, cnt_o,
              xv, yv, zv, bxv, byv, bzv, pnv, cxv, cyv, czv,
              rb1, rb2, rb3, c1v, c2v, c3v,
              idx1, idx2, idx3, idxg, idxg2, rows1, rows2, rows3,
              rs1, rs2, rs3, gs1, gs2, gs3, os1, os2, os3):
    branches = list(zip(_RADII2, _NS2, _NSP2))
    bs = _B * _NP2
    bpw = bs // _NW
    nchunks = _NP1 // 16
    wid = lax.axis_index("s") * _NC + lax.axis_index("c")
    b = wid // (_NP2 // bpw)
    for c, dst in enumerate([xv, yv, zv]):
        pltpu.sync_copy(x_hbm.at[pl.ds((c * _B + b) * _NP1, _NP1)], dst)
    for c, dst in enumerate([cxv, cyv, czv]):
        pltpu.sync_copy(cent_hbm.at[pl.ds(c * bs + wid * bpw, bpw)], dst)
    _precompute_rounded(xv, yv, zv, bxv, byv, bzv, pnv, _NP1)
    relbufs = [rb1, rb2, rb3]
    relsems = [rs1, rs2, rs3]
    gsems = [gs1, gs2, gs3]
    osems = [os1, os2, os3]
    for j, (r, ns, nsp) in enumerate(branches):
        _zero_f32(relbufs[j], 3 * nsp)
    lx = xv[pl.ds(_NP1 - 16, 16)][15]
    ly = yv[pl.ds(_NP1 - 16, 16)][15]
    lz = zv[pl.ds(_NP1 - 16, 16)][15]
    rel_outs = [rel1_o, rel2_o, rel3_o]
    g_outs = [g1_o, g2_o, g3_o]
    f_tabs = [f1_hbm, f2_hbm, f3_hbm]
    cnt_bufs = [c1v, c2v, c3v]
    idxbufs = [idx1, idx2, idx3]
    rowbufs = [rows1, rows2, rows3]
    zeros16 = jnp.zeros((16,), jnp.int32)

    def fire_gather(j, nsp, ib):
        if nsp <= 128:
            pltpu.async_copy(f_tabs[j].at[ib], rowbufs[j], gsems[j])
        else:
            for q in range(8):
                idxg[pl.ds(q * 16, 16)] = ib[pl.ds(q * 16, 16)]
            pltpu.async_copy(f_tabs[j].at[idxg],
                             rowbufs[j].at[pl.ds(0, 128)], gsems[j])
            for q in range(8, nsp // 16):
                idxg2[pl.ds((q - 8) * 16, 16)] = ib[pl.ds(q * 16, 16)]
            pltpu.async_copy(f_tabs[j].at[idxg2],
                             rowbufs[j].at[pl.ds(128, nsp - 128)], gsems[j])

    def wait_gather(j, nsp, ib):
        if nsp <= 128:
            pltpu.make_async_copy(f_tabs[j].at[ib], rowbufs[j],
                                  gsems[j]).wait()
        else:
            pltpu.make_async_copy(f_tabs[j].at[idxg],
                                  rowbufs[j].at[pl.ds(0, 128)],
                                  gsems[j]).wait()
            pltpu.make_async_copy(f_tabs[j].at[idxg2],
                                  rowbufs[j].at[pl.ds(128, nsp - 128)],
                                  gsems[j]).wait()

    def per_ball(i, carry):
        cx = _centroid_scalar(cxv, i)
        cy = _centroid_scalar(cyv, i)
        cz = _centroid_scalar(czv, i)
        for j, (r, ns, nsp) in enumerate(branches):
            ib = idxbufs[j]
            relbuf = relbufs[j]

            @pl.when(i > 0)
            def _():
                pltpu.make_async_copy(
                    relbuf.at[pl.ds(0, 3 * nsp)],
                    rel_outs[j].at[pl.ds((wid * bpw + i - 1) * 3 * nsp,
                                         3 * nsp)],
                    relsems[j]).wait()
                pltpu.make_async_copy(
                    rowbufs[j],
                    g_outs[j].at[pl.ds((wid * bpw + i - 1) * nsp, nsp)],
                    osems[j]).wait()

            for q in range(nsp // 16):
                ib[pl.ds(q * 16, 16)] = zeros16
            cnt = _bq_scan([xv, yv, zv], [bxv, byv, bzv], pnv, cx, cy, cz,
                           jnp.float32(r * r), ns, nsp, nchunks,
                           relbuf, ib, b * _NP1)

            @pl.when(cnt == 0)
            def _():
                lane = lax.iota(jnp.int32, 16)
                v = jnp.where(lane == 0, lx - cx,
                              jnp.where(lane == 1, ly - cy, lz - cz))
                relbuf[pl.ds(0, 16)] = v
                ib[pl.ds(0, 16)] = jnp.full((16,), b * _NP1 + _NP1 - 1,
                                            jnp.int32)

            _store_scalar_i32(cnt_bufs[j], i,
                              jnp.maximum(jnp.minimum(cnt, ns), 1))
            fire_gather(j, nsp, ib)
            pltpu.async_copy(relbuf.at[pl.ds(0, 3 * nsp)],
                             rel_outs[j].at[pl.ds((wid * bpw + i) * 3 * nsp,
                                                  3 * nsp)],
                             relsems[j])
        for j, (r, ns, nsp) in enumerate(branches):
            wait_gather(j, nsp, idxbufs[j])
            pltpu.async_copy(rowbufs[j],
                             g_outs[j].at[pl.ds((wid * bpw + i) * nsp, nsp)],
                             osems[j])
        return carry

    lax.fori_loop(0, bpw, per_ball, 0)
    for j, (r, ns, nsp) in enumerate(branches):
        pltpu.make_async_copy(
            relbufs[j].at[pl.ds(0, 3 * nsp)],
            rel_outs[j].at[pl.ds((wid * bpw + bpw - 1) * 3 * nsp, 3 * nsp)],
            relsems[j]).wait()
        pltpu.make_async_copy(
            rowbufs[j],
            g_outs[j].at[pl.ds((wid * bpw + bpw - 1) * nsp, nsp)],
            osems[j]).wait()
    for j in range(3):
        pltpu.sync_copy(cnt_bufs[j], cnt_o.at[pl.ds(j * bs + wid * bpw, bpw)])


def _ballquery2(xyz_flat, cent_flat, f1, f2, f3):
    bs = _B * _NP2
    bpw = bs // _NW
    mesh = plsc.VectorSubcoreMesh(core_axis_name="c", subcore_axis_name="s")
    outs = (tuple(jax.ShapeDtypeStruct((bs * 3 * nsp,), jnp.float32)
                  for nsp in _NSP2)
            + tuple(jax.ShapeDtypeStruct((bs * nsp, _DG), jnp.float32)
                    for nsp in _NSP2)
            + (jax.ShapeDtypeStruct((3 * bs,), jnp.int32),))
    k = pl.kernel(
        _bq2_body,
        out_type=outs,
        mesh=mesh,
        compiler_params=pltpu.CompilerParams(needs_layout_passes=False),
        scratch_types=[
            pltpu.VMEM((_NP1,), jnp.float32),
            pltpu.VMEM((_NP1,), jnp.float32),
            pltpu.VMEM((_NP1,), jnp.float32),
            pltpu.VMEM((_NP1,), jnp.float32),
            pltpu.VMEM((_NP1,), jnp.float32),
            pltpu.VMEM((_NP1,), jnp.float32),
            pltpu.VMEM((_NP1,), jnp.float32),
            pltpu.VMEM((bpw,), jnp.float32),
            pltpu.VMEM((bpw,), jnp.float32),
            pltpu.VMEM((bpw,), jnp.float32),
            pltpu.VMEM((3 * _NSP2[0],), jnp.float32),
            pltpu.VMEM((3 * _NSP2[1],), jnp.float32),
            pltpu.VMEM((3 * _NSP2[2],), jnp.float32),
            pltpu.VMEM((bpw,), jnp.int32),
            pltpu.VMEM((bpw,), jnp.int32),
            pltpu.VMEM((bpw,), jnp.int32),
            pltpu.VMEM((_NSP2[0],), jnp.int32),
            pltpu.VMEM((_NSP2[1],), jnp.int32),
            pltpu.VMEM((_NSP2[2],), jnp.int32),
            pltpu.VMEM((128,), jnp.int32),
            pltpu.VMEM((16,), jnp.int32),
            pltpu.VMEM((_NSP2[0], _DG), jnp.float32),
            pltpu.VMEM((_NSP2[1], _DG), jnp.float32),
            pltpu.VMEM((_NSP2[2], _DG), jnp.float32),
            pltpu.SemaphoreType.DMA,
            pltpu.SemaphoreType.DMA,
            pltpu.SemaphoreType.DMA,
            pltpu.SemaphoreType.DMA,
            pltpu.SemaphoreType.DMA,
            pltpu.SemaphoreType.DMA,
            pltpu.SemaphoreType.DMA,
            pltpu.SemaphoreType.DMA,
            pltpu.SemaphoreType.DMA,
        ],
    )
    return k(xyz_flat, cent_flat, f1, f2, f3)


def _mlp_pool_body(sb, nsp, has_f, *refs):
    if has_f:
        x_ref, f_ref, cnt_ref = refs[:3]
        w_refs = refs[3:-1]
    else:
        x_ref, cnt_ref = refs[:2]
        w_refs = refs[2:-1]
    o_ref = refs[-1]
    h = jnp.dot(x_ref[...], w_refs[0][...], preferred_element_type=jnp.float32)
    if has_f:
        h = h + f_ref[...]
    h = jax.nn.relu(h + w_refs[1][...])
    nl = (len(w_refs) - 2) // 2
    for l in range(nl):
        h = jnp.dot(h, w_refs[2 + 2 * l][...],
                    preferred_element_type=jnp.float32)
        h = jax.nn.relu(h + w_refs[3 + 2 * l][...])
    c3 = h.shape[-1]
    h = h.reshape(sb, nsp, c3)
    k_iota = lax.broadcasted_iota(jnp.int32, (sb, nsp, 1), 1)
    h = jnp.where(k_iota < cnt_ref[...][:, :, None], h, jnp.float32(-1e30))
    o_ref[...] = jnp.max(h, axis=1)


def _mlp_pool(x, f, cnt, ws, nsp, sb):
    bs = cnt.shape[0]
    c3 = ws[-1].shape[-1]
    grid = (bs // sb,)
    in_specs = [pl.BlockSpec((sb * nsp, 3), lambda i: (i, 0))]
    args = [x]
    if f is not None:
        in_specs.append(pl.BlockSpec((sb * nsp, f.shape[-1]),
                                     lambda i: (i, 0)))
        args.append(f)
    in_specs.append(pl.BlockSpec((sb, 1), lambda i: (i, 0)))
    args.append(cnt)
    for w in ws:
        in_specs.append(pl.BlockSpec(w.shape, lambda i: (0,) * w.ndim))
        args.append(w)
    return pl.pallas_call(
        functools.partial(_mlp_pool_body, sb, nsp, f is not None),
        grid=grid,
        in_specs=in_specs,
        out_specs=pl.BlockSpec((sb, c3), lambda i: (i, 0)),
        out_shape=jax.ShapeDtypeStruct((bs, c3), jnp.float32),
    )(*args)


def _matmul_body(x_ref, w_ref, o_ref):
    o_ref[...] = jnp.dot(x_ref[...], w_ref[...],
                         preferred_element_type=jnp.float32)


def _matmul(x, w, rb):
    m, kdim = x.shape
    n = w.shape[1]
    return pl.pallas_call(
        _matmul_body,
        grid=(m // rb,),
        in_specs=[pl.BlockSpec((rb, kdim), lambda i: (i, 0)),
                  pl.BlockSpec((kdim, n), lambda i: (0, 0))],
        out_specs=pl.BlockSpec((rb, n), lambda i: (i, 0)),
        out_shape=jax.ShapeDtypeStruct((m, n), jnp.float32),
    )(x, w)


def _sa_body(nb, x_ref, w0, b0, w1, b1, w2, b2, o_ref):
    h = jax.nn.relu(jnp.dot(x_ref[...], w0[...],
                            preferred_element_type=jnp.float32) + b0[...])
    h = jax.nn.relu(jnp.dot(h, w1[...],
                            preferred_element_type=jnp.float32) + b1[...])
    h = jax.nn.relu(jnp.dot(h, w2[...],
                            preferred_element_type=jnp.float32) + b2[...])
    h = h.reshape(nb, _NP2, h.shape[-1])
    o_ref[...] = jnp.max(h, axis=1)


def _sa_final(x, ws, nb):
    c3 = ws[-1].shape[-1]
    in_specs = [pl.BlockSpec((nb * _NP2, x.shape[1]), lambda i: (i, 0))]
    for w in ws:
        in_specs.append(pl.BlockSpec(w.shape, lambda i: (0,) * w.ndim))
    return pl.pallas_call(
        functools.partial(_sa_body, nb),
        grid=(_B // nb,),
        in_specs=in_specs,
        out_specs=pl.BlockSpec((nb, c3), lambda i: (i, 0)),
        out_shape=jax.ShapeDtypeStruct((_B, c3), jnp.float32),
    )(x, *ws)


def _fold(params, prefix, nl):
    ws = []
    for l in range(nl):
        w = params[prefix + "_l%d_W" % l]
        g = params[prefix + "_l%d_g" % l]
        b = params[prefix + "_l%d_b" % l]
        ws.append(w * (g * _EPS_SCALE)[None, :])
        ws.append(b[None, :])
    return ws


def kernel(pc, params):
    xyz3 = jnp.transpose(pc, (2, 0, 1))

    new_xyz1, cent3_1, xyz3_1 = _fps(xyz3.reshape(3 * _B, _N1), _NP1)
    rel1, rel2, rel3, cnt1 = _ballquery1(xyz3.reshape(-1),
                                         cent3_1.reshape(-1))
    bs1 = _B * _NP1
    feats = []
    for j, (rel, nsp, sb) in enumerate(
            zip([rel1, rel2, rel3], _NSP1, [64, 64, 32])):
        relr = rel.reshape(-1, 3)
        cntj = cnt1[j * bs1:(j + 1) * bs1].reshape(bs1, 1)
        ws = _fold(params, "msg1_b%d" % j, 3)
        feats.append(_mlp_pool(relr, None, cntj, ws, nsp, sb))
    feat1 = jnp.concatenate(feats, axis=-1)

    w2 = [_fold(params, "msg2_b%d" % j, 3) for j in range(3)]
    wcat = jnp.concatenate(
        [w2[0][0][3:], jnp.zeros((320, _DG - _D2[0]), jnp.float32),
         w2[1][0][3:], w2[2][0][3:]], axis=1)
    ftab = _matmul(feat1, wcat, 1024)
    f1, f2, f3 = (ftab[:, :_DG], ftab[:, _DG:2 * _DG], ftab[:, 2 * _DG:])

    new_xyz2, cent3_2, _ = _fps(xyz3_1.reshape(3 * _B, _NP1), _NP2)
    (srel1, srel2, srel3, g1, g2, g3, cnt2) = _ballquery2(
        xyz3_1.reshape(-1), cent3_2.reshape(-1), f1, f2, f3)
    bs2 = _B * _NP2
    feats2 = []
    for j, (rel, g, nsp, sb) in enumerate(
            zip([srel1, srel2, srel3], [g1, g2, g3], _NSP2, [32, 32, 16])):
        relr = rel.reshape(-1, 3)
        fg = g[:, :_D2[j]]
        cntj = cnt2[j * bs2:(j + 1) * bs2].reshape(bs2, 1)
        ws = w2[j]
        wsj = [ws[0][:3]] + ws[1:]
        feats2.append(_mlp_pool(relr, fg, cntj, wsj, nsp, sb))
    feat2 = jnp.concatenate(feats2, axis=-1)

    x2 = jnp.concatenate([new_xyz2.reshape(bs2, 3), feat2], axis=-1)
    sa_ws = _fold(params, "sa", 3)
    return _sa_final(x2, sa_ws, 8)
